# baseline (speedup 1.0000x reference)
import functools

import numpy as np

import jax
import jax.numpy as jnp
from jax import lax
from jax.experimental import pallas as pl
from jax.experimental.pallas import tpu as pltpu
from jax.experimental.pallas import tpu_sc as plsc

N = 10000
D = 128
DIM_M = 64
D_MAX = 1000 + 512
OMEGA = 0.1
K_FAIR = 2.0
N_IDX = 1024

BM = 400
GRID_M = N // BM
BM2 = 1000
GRID_M2 = N // BM2

NW = 32
B_PE = 10240
PE_PER_W = B_PE // NW
IDX_PER_W = N_IDX // NW


def _build_pe_table():
    pos = np.arange(D_MAX, dtype=np.float64)[:, None]
    ii = np.arange(DIM_M, dtype=np.float64)[None, :]
    pe = pos / np.power(10000.0, (ii - (ii % 2)) / DIM_M)
    pe[:, 0::2] = np.sin(pe[:, 0::2])
    pe[:, 1::2] = np.cos(pe[:, 1::2])
    out = np.zeros((D_MAX, 2 * DIM_M), dtype=np.float32)
    out[:, :DIM_M] = pe
    return out


_PE_TABLE = _build_pe_table()

_F32 = jnp.float32


def _builder_body(x_ref, deg_ref, wconv_ref, wg11_ref, wg12_ref, wg21_ref,
                  wg22_ref, wb11_ref, wb12_ref, wb21_ref, wb22_ref,
                  c_ref, hh_ref, agt_ref, bgt_ref, abt_ref, bbt_ref,
                  kthr_ref):
    x = x_ref[...]
    degf = deg_ref[...].astype(_F32)
    h = lax.dot_general(x, wconv_ref[...], (((1,), (1,)), ((), ())),
                        preferred_element_type=_F32)
    dinv = lax.rsqrt(degf + 1.0)
    hh = dinv * h
    hh_ref[...] = hh
    c_ref[...] = jnp.concatenate([x, hh, h], axis=1).astype(jnp.bfloat16)

    def fold(a_ref, b_ref):
        return lax.dot_general(a_ref[...], b_ref[...],
                               (((0,), (1,)), ((), ())),
                               preferred_element_type=_F32)

    agt_ref[...] = fold(wg11_ref, wg12_ref)
    bgt_ref[...] = fold(wg21_ref, wg22_ref)
    abt_ref[...] = fold(wb11_ref, wb12_ref)
    bbt_ref[...] = fold(wb21_ref, wb22_ref)
    kthr_ref[...] = jnp.sum(degf).reshape(1, 1) * (K_FAIR / N)


def _main_body(adj_ref, x_ref, hh_ref, deg_ref, c_ref,
               rrel_ref, agt_ref, bgt_ref, abt_ref,
               bbt_ref, wconv_ref, wadd_ref, wrev_ref,
               outp_ref, rel_ref, tadd_ref, trev_ref):
    adj_bf = adj_ref[...].astype(jnp.bfloat16)
    s = jnp.dot(adj_bf, c_ref[...], preferred_element_type=_F32)

    x = x_ref[...]
    hh = hh_ref[...]
    degf = deg_ref[...].astype(_F32)
    inv_d1 = 1.0 / (degf + 1.0)
    neighbor = s[:, :D] * inv_d1
    s_hh = s[:, D:2 * D]
    s_h = s[:, 2 * D:]

    def mm(a, w_ref):
        return jnp.dot(a, w_ref[...], preferred_element_type=_F32)

    def mmt(a, w_ref):
        return lax.dot_general(a, w_ref[...], (((1,), (1,)), ((), ())),
                               preferred_element_type=_F32)

    pre_g = mm(x, agt_ref) + mm(neighbor, bgt_ref)
    pre_b = mm(x, abt_ref) + mm(neighbor, bbt_ref)
    gamma_r = jnp.where(pre_g >= 0.0, pre_g, 0.2 * pre_g) + 1.0
    beta_r = jnp.where(pre_b >= 0.0, pre_b, 0.2 * pre_b)
    rel = x + gamma_r * rrel_ref[...] + beta_r - neighbor
    rel_ref[...] = rel

    dinv = lax.rsqrt(degf + 1.0)
    miss = mmt(rel, wconv_ref) * inv_d1
    outp_ref[...] = dinv * (s_hh + hh) + miss

    i_agg = jnp.where(degf == 0.0, 0.0,
                      (s_h * (DIM_M ** 0.5)) / jnp.maximum(degf, 1.0))
    tadd_ref[...] = mmt(i_agg, wadd_ref)
    trev_ref[...] = mmt(i_agg, wrev_ref)


def _film_body(mdv_ref, deg_ref, tadd_ref, trev_ref, outp_ref, kthr_ref,
               bgam_ref, bbet_ref, wgam_ref, wbet_ref,
               out_ref, nbg_ref):
    degf = deg_ref[...].astype(_F32)
    mdv = mdv_ref[...]

    def mm(a, w_ref):
        return jnp.dot(a, w_ref[...], preferred_element_type=_F32)

    pg = mm(mdv, wgam_ref) + bgam_ref[...]
    pb = mm(mdv, wbet_ref) + bbet_ref[...]
    gamma = jnp.where(pg >= 0.0, pg, 0.01 * pg)
    beta = jnp.where(pb >= 0.0, pb, 0.01 * pb)

    gp1 = gamma + 1.0
    b_add = gp1 * tadd_ref[...] + beta
    b_rev = gp1 * trev_ref[...] + beta

    r_mask = jnp.where(degf < kthr_ref[...], 1.0, 0.0)

    def rownorm(t):
        return jnp.sqrt(jnp.sum(t * t, axis=1, keepdims=True))

    nb = r_mask * rownorm(b_add) + (1.0 - r_mask) * rownorm(b_rev)
    ng = rownorm(gamma) + rownorm(beta)
    nbg_ref[...] = jnp.concatenate(
        [nb, ng, jnp.zeros((BM2, D - 2), _F32)], axis=1)

    bias = OMEGA * (r_mask * b_add - (1.0 - r_mask) * b_rev)
    out_ref[...] = outp_ref[...] + bias


def _make_pe_gather():
    mesh = plsc.VectorSubcoreMesh(core_axis_name="c", subcore_axis_name="s")
    info = plsc.get_sparse_core_info()
    nc = info.num_cores

    @functools.partial(
        pl.kernel, mesh=mesh,
        out_type=jax.ShapeDtypeStruct((B_PE, 2 * DIM_M), _F32),
        scratch_types=[
            pltpu.VMEM((PE_PER_W,), jnp.int32),
            pltpu.VMEM((PE_PER_W, 2 * DIM_M), _F32),
            pltpu.SemaphoreType.DMA,
        ],
    )
    def pe_gather(table_hbm, idx_hbm, out_hbm, idx_v, rows_v, sem):
        wid = lax.axis_index("s") * nc + lax.axis_index("c")
        base = wid * PE_PER_W
        pltpu.sync_copy(idx_hbm.at[pl.ds(base, PE_PER_W)], idx_v)
        pltpu.async_copy(table_hbm.at[idx_v], rows_v, sem).wait()
        pltpu.sync_copy(rows_v, out_hbm.at[pl.ds(base, PE_PER_W)])

    return pe_gather


def _make_loss_gather():
    mesh = plsc.VectorSubcoreMesh(core_axis_name="c", subcore_axis_name="s")
    info = plsc.get_sparse_core_info()
    nc = info.num_cores

    @functools.partial(
        pl.kernel, mesh=mesh,
        out_type=jax.ShapeDtypeStruct((NW, D), _F32),
        scratch_types=[
            pltpu.VMEM((IDX_PER_W,), jnp.int32),
            pltpu.VMEM((IDX_PER_W, D), _F32),
            pltpu.VMEM((D,), _F32),
            pltpu.SemaphoreType.DMA,
        ],
    )
    def loss_gather(nbg_hbm, idx_hbm, out_hbm, idx_v, val_v, acc_v, sem):
        wid = lax.axis_index("s") * nc + lax.axis_index("c")
        base = wid * IDX_PER_W
        pltpu.sync_copy(idx_hbm.at[pl.ds(base, IDX_PER_W)], idx_v)
        pltpu.async_copy(nbg_hbm.at[idx_v], val_v, sem).wait()
        acc = val_v[0, pl.ds(0, 16)]
        for i in range(1, IDX_PER_W):
            acc = acc + val_v[i, pl.ds(0, 16)]
        acc_v[pl.ds(0, 16)] = acc
        zeros16 = jnp.zeros((16,), _F32)
        for j in range(16, D, 16):
            acc_v[pl.ds(j, 16)] = zeros16
        pltpu.sync_copy(acc_v, out_hbm.at[wid])

    return loss_gather


def kernel(x, adj, degree, idx, edge, head, Wg11, Wg12, Wg21, Wg22, Wb11,
           Wb12, Wb21, Wb22, r_rel, W_conv, W_gamma, W_beta, b_gamma,
           b_beta, W_add, W_rev):
    del edge, head
    degree = degree.astype(jnp.int32)

    f128 = jax.ShapeDtypeStruct((D, D), _F32)
    c_shape, hh_shape, agt, bgt, abt, bbt, kthr = pl.pallas_call(
        _builder_body,
        out_shape=[
            jax.ShapeDtypeStruct((N, 3 * D), jnp.bfloat16),
            jax.ShapeDtypeStruct((N, D), _F32),
            f128, f128, f128, f128,
            jax.ShapeDtypeStruct((1, 1), _F32),
        ],
    )(x, degree, W_conv, Wg11, Wg12, Wg21, Wg22, Wb11, Wb12, Wb21, Wb22)
    c_bf, hh = c_shape, hh_shape

    pe_tab = jnp.asarray(_PE_TABLE)
    deg_idx = jnp.minimum(degree[:, 0], D_MAX - 1)
    deg_idx = jnp.concatenate(
        [deg_idx, jnp.zeros((B_PE - N,), jnp.int32)])
    mdv_pad = _make_pe_gather()(pe_tab, deg_idx)
    wgam_pad = jnp.pad(W_gamma, ((0, D - DIM_M), (0, 0)))
    wbet_pad = jnp.pad(W_beta, ((0, D - DIM_M), (0, 0)))

    blk = lambda shape: pl.BlockSpec(shape, lambda i: (i, 0))
    cst = lambda shape: pl.BlockSpec(shape, lambda i: (0, 0))
    outp, rel, t_add, t_rev = pl.pallas_call(
        _main_body,
        grid=(GRID_M,),
        in_specs=[
            blk((BM, N)),
            blk((BM, D)),
            blk((BM, D)),
            blk((BM, 1)),
            cst((N, 3 * D)),
            cst((1, D)),
            cst((D, D)),
            cst((D, D)),
            cst((D, D)),
            cst((D, D)),
            cst((D, D)),
            cst((D, D)),
            cst((D, D)),
        ],
        out_specs=[
            blk((BM, D)),
            blk((BM, D)),
            blk((BM, D)),
            blk((BM, D)),
        ],
        out_shape=[
            jax.ShapeDtypeStruct((N, D), _F32),
            jax.ShapeDtypeStruct((N, D), _F32),
            jax.ShapeDtypeStruct((N, D), _F32),
            jax.ShapeDtypeStruct((N, D), _F32),
        ],
        compiler_params=pltpu.CompilerParams(
            dimension_semantics=("arbitrary",)),
    )(adj, x, hh, degree, c_bf, r_rel,
      agt, bgt, abt, bbt, W_conv, W_add, W_rev)

    blk2 = lambda shape: pl.BlockSpec(shape, lambda i: (i, 0))
    out, nbg = pl.pallas_call(
        _film_body,
        grid=(GRID_M2,),
        in_specs=[
            blk2((BM2, D)),
            blk2((BM2, 1)),
            blk2((BM2, D)),
            blk2((BM2, D)),
            blk2((BM2, D)),
            cst((1, 1)),
            cst((1, D)),
            cst((1, D)),
            cst((D, D)),
            cst((D, D)),
        ],
        out_specs=[
            blk2((BM2, D)),
            blk2((BM2, D)),
        ],
        out_shape=[
            jax.ShapeDtypeStruct((N, D), _F32),
            jax.ShapeDtypeStruct((N, D), _F32),
        ],
        compiler_params=pltpu.CompilerParams(
            dimension_semantics=("arbitrary",)),
    )(mdv_pad, degree, t_add, t_rev, outp, kthr, b_gamma, b_beta,
      wgam_pad, wbet_pad)

    partials = _make_loss_gather()(nbg, idx.astype(jnp.int32))
    l_b = jnp.sum(partials[:, 0]) / N_IDX
    l_film = jnp.sum(partials[:, 1]) / N_IDX

    return out, l_b, l_film, rel


# speedup vs baseline: 1.1263x; 1.1263x over previous
import functools

import numpy as np

import jax
import jax.numpy as jnp
from jax import lax
from jax.experimental import pallas as pl
from jax.experimental.pallas import tpu as pltpu
from jax.experimental.pallas import tpu_sc as plsc

N = 10000
D = 128
DIM_M = 64
D_MAX = 1000 + 512
OMEGA = 0.1
K_FAIR = 2.0
N_IDX = 1024

BM = 400
GRID_M = N // BM

NW = 32
B_PE = 10240
PE_PER_W = B_PE // NW
IDX_PER_W = N_IDX // NW


def _build_pe_table():
    pos = np.arange(D_MAX, dtype=np.float64)[:, None]
    ii = np.arange(DIM_M, dtype=np.float64)[None, :]
    pe = pos / np.power(10000.0, (ii - (ii % 2)) / DIM_M)
    pe[:, 0::2] = np.sin(pe[:, 0::2])
    pe[:, 1::2] = np.cos(pe[:, 1::2])
    out = np.zeros((D_MAX, 2 * DIM_M), dtype=np.float32)
    out[:, :DIM_M] = pe
    return out


_PE_TABLE = _build_pe_table()

_F32 = jnp.float32


def _builder_body(x_ref, deg_ref, wg11_ref, wg12_ref, wg21_ref,
                  wg22_ref, wb11_ref, wb12_ref, wb21_ref, wb22_ref,
                  c_ref, agt_ref, bgt_ref, abt_ref, bbt_ref,
                  kthr_ref):
    x = x_ref[...]
    degf = deg_ref[...].astype(_F32)
    dinv = lax.rsqrt(degf + 1.0)
    c_ref[...] = jnp.concatenate([x, dinv * x], axis=1).astype(jnp.bfloat16)

    def fold(a_ref, b_ref):
        return lax.dot_general(a_ref[...], b_ref[...],
                               (((0,), (1,)), ((), ())),
                               preferred_element_type=_F32)

    agt_ref[...] = fold(wg11_ref, wg12_ref)
    bgt_ref[...] = fold(wg21_ref, wg22_ref)
    abt_ref[...] = fold(wb11_ref, wb12_ref)
    bbt_ref[...] = fold(wb21_ref, wb22_ref)
    kthr_ref[...] = jnp.sum(degf).reshape(1, 1) * (K_FAIR / N)


def _main_body(adj_ref, x_ref, deg_ref, mdv_ref, c_ref,
               rrel_ref, agt_ref, bgt_ref, abt_ref, bbt_ref,
               wconv_ref, wadd_ref, wrev_ref, kthr_ref,
               bgam_ref, bbet_ref, wgam_ref, wbet_ref,
               out_ref, rel_ref, nbg_ref):
    adj_bf = adj_ref[...].astype(jnp.bfloat16)
    s = jnp.dot(adj_bf, c_ref[...], preferred_element_type=_F32)

    x = x_ref[...]
    degf = deg_ref[...].astype(_F32)
    inv_d1 = 1.0 / (degf + 1.0)
    dinv = lax.rsqrt(degf + 1.0)
    axp = s[:, :D]
    adx = s[:, D:]
    neighbor = axp * inv_d1

    def mm(a, w_ref):
        return jnp.dot(a, w_ref[...], preferred_element_type=_F32)

    def mmt(a, w_ref):
        return lax.dot_general(a, w_ref[...], (((1,), (1,)), ((), ())),
                               preferred_element_type=_F32)

    pre_g = mm(x, agt_ref) + mm(neighbor, bgt_ref)
    pre_b = mm(x, abt_ref) + mm(neighbor, bbt_ref)
    gamma_r = jnp.where(pre_g >= 0.0, pre_g, 0.2 * pre_g) + 1.0
    beta_r = jnp.where(pre_b >= 0.0, pre_b, 0.2 * pre_b)
    rel = x + gamma_r * rrel_ref[...] + beta_r - neighbor
    rel_ref[...] = rel

    s_hh = mmt(adx, wconv_ref)
    hh = dinv * mmt(x, wconv_ref)
    miss = mmt(rel, wconv_ref) * inv_d1
    out_gcn = dinv * (s_hh + hh) + miss

    s_h = mmt(axp, wconv_ref)
    i_agg = jnp.where(degf == 0.0, 0.0,
                      (s_h * (DIM_M ** 0.5)) / jnp.maximum(degf, 1.0))
    t_add = mmt(i_agg, wadd_ref)
    t_rev = mmt(i_agg, wrev_ref)

    pg = mm(mdv_ref[...], wgam_ref) + bgam_ref[...]
    pb = mm(mdv_ref[...], wbet_ref) + bbet_ref[...]
    gamma = jnp.where(pg >= 0.0, pg, 0.01 * pg)
    beta = jnp.where(pb >= 0.0, pb, 0.01 * pb)

    gp1 = gamma + 1.0
    b_add = gp1 * t_add + beta
    b_rev = gp1 * t_rev + beta

    r_mask = jnp.where(degf < kthr_ref[...], 1.0, 0.0)

    def rownorm(t):
        return jnp.sqrt(jnp.sum(t * t, axis=1, keepdims=True))

    nb = r_mask * rownorm(b_add) + (1.0 - r_mask) * rownorm(b_rev)
    ng = rownorm(gamma) + rownorm(beta)
    nbg_ref[...] = jnp.concatenate(
        [nb, ng, jnp.zeros((BM, D - 2), _F32)], axis=1)

    bias = OMEGA * (r_mask * b_add - (1.0 - r_mask) * b_rev)
    out_ref[...] = out_gcn + bias


def _make_pe_gather():
    mesh = plsc.VectorSubcoreMesh(core_axis_name="c", subcore_axis_name="s")
    info = plsc.get_sparse_core_info()
    nc = info.num_cores

    @functools.partial(
        pl.kernel, mesh=mesh,
        out_type=jax.ShapeDtypeStruct((B_PE, 2 * DIM_M), _F32),
        scratch_types=[
            pltpu.VMEM((PE_PER_W,), jnp.int32),
            pltpu.VMEM((PE_PER_W, 2 * DIM_M), _F32),
            pltpu.SemaphoreType.DMA,
        ],
    )
    def pe_gather(table_hbm, idx_hbm, out_hbm, idx_v, rows_v, sem):
        wid = lax.axis_index("s") * nc + lax.axis_index("c")
        base = wid * PE_PER_W
        pltpu.sync_copy(idx_hbm.at[pl.ds(base, PE_PER_W)], idx_v)
        pltpu.async_copy(table_hbm.at[idx_v], rows_v, sem).wait()
        pltpu.sync_copy(rows_v, out_hbm.at[pl.ds(base, PE_PER_W)])

    return pe_gather


def _make_loss_gather():
    mesh = plsc.VectorSubcoreMesh(core_axis_name="c", subcore_axis_name="s")
    info = plsc.get_sparse_core_info()
    nc = info.num_cores

    @functools.partial(
        pl.kernel, mesh=mesh,
        out_type=jax.ShapeDtypeStruct((NW, D), _F32),
        scratch_types=[
            pltpu.VMEM((IDX_PER_W,), jnp.int32),
            pltpu.VMEM((IDX_PER_W, D), _F32),
            pltpu.VMEM((D,), _F32),
            pltpu.SemaphoreType.DMA,
        ],
    )
    def loss_gather(nbg_hbm, idx_hbm, out_hbm, idx_v, val_v, acc_v, sem):
        wid = lax.axis_index("s") * nc + lax.axis_index("c")
        base = wid * IDX_PER_W
        pltpu.sync_copy(idx_hbm.at[pl.ds(base, IDX_PER_W)], idx_v)
        pltpu.async_copy(nbg_hbm.at[idx_v], val_v, sem).wait()
        acc = val_v[0, pl.ds(0, 16)]
        for i in range(1, IDX_PER_W):
            acc = acc + val_v[i, pl.ds(0, 16)]
        acc_v[pl.ds(0, 16)] = acc
        zeros16 = jnp.zeros((16,), _F32)
        for j in range(16, D, 16):
            acc_v[pl.ds(j, 16)] = zeros16
        pltpu.sync_copy(acc_v, out_hbm.at[wid])

    return loss_gather


def kernel(x, adj, degree, idx, edge, head, Wg11, Wg12, Wg21, Wg22, Wb11,
           Wb12, Wb21, Wb22, r_rel, W_conv, W_gamma, W_beta, b_gamma,
           b_beta, W_add, W_rev):
    del edge, head
    degree = degree.astype(jnp.int32)

    f128 = jax.ShapeDtypeStruct((D, D), _F32)
    c_bf, agt, bgt, abt, bbt, kthr = pl.pallas_call(
        _builder_body,
        out_shape=[
            jax.ShapeDtypeStruct((N, 2 * D), jnp.bfloat16),
            f128, f128, f128, f128,
            jax.ShapeDtypeStruct((1, 1), _F32),
        ],
    )(x, degree, Wg11, Wg12, Wg21, Wg22, Wb11, Wb12, Wb21, Wb22)

    pe_tab = jnp.asarray(_PE_TABLE)
    deg_idx = jnp.minimum(degree[:, 0], D_MAX - 1)
    deg_idx = jnp.concatenate(
        [deg_idx, jnp.zeros((B_PE - N,), jnp.int32)])
    mdv_pad = _make_pe_gather()(pe_tab, deg_idx)
    wgam_pad = jnp.pad(W_gamma, ((0, D - DIM_M), (0, 0)))
    wbet_pad = jnp.pad(W_beta, ((0, D - DIM_M), (0, 0)))

    blk = lambda shape: pl.BlockSpec(shape, lambda i: (i, 0))
    cst = lambda shape: pl.BlockSpec(shape, lambda i: (0, 0))
    out, rel, nbg = pl.pallas_call(
        _main_body,
        grid=(GRID_M,),
        in_specs=[
            blk((BM, N)),
            blk((BM, D)),
            blk((BM, 1)),
            blk((BM, D)),
            cst((N, 2 * D)),
            cst((1, D)),
            cst((D, D)),
            cst((D, D)),
            cst((D, D)),
            cst((D, D)),
            cst((D, D)),
            cst((D, D)),
            cst((D, D)),
            cst((1, 1)),
            cst((1, D)),
            cst((1, D)),
            cst((D, D)),
            cst((D, D)),
        ],
        out_specs=[
            blk((BM, D)),
            blk((BM, D)),
            blk((BM, D)),
        ],
        out_shape=[
            jax.ShapeDtypeStruct((N, D), _F32),
            jax.ShapeDtypeStruct((N, D), _F32),
            jax.ShapeDtypeStruct((N, D), _F32),
        ],
        compiler_params=pltpu.CompilerParams(
            dimension_semantics=("arbitrary",)),
    )(adj, x, degree, mdv_pad[:N], c_bf, r_rel,
      agt, bgt, abt, bbt, W_conv, W_add, W_rev, kthr,
      b_gamma, b_beta, wgam_pad, wbet_pad)

    partials = _make_loss_gather()(nbg, idx.astype(jnp.int32))
    l_b = jnp.sum(partials[:, 0]) / N_IDX
    l_film = jnp.sum(partials[:, 1]) / N_IDX

    return out, l_b, l_film, rel
